# Initial kernel scaffold; baseline (speedup 1.0000x reference)
import jax
import jax.numpy as jnp
from jax.experimental import pallas as pl


def kernel(x, edge_index, edge_attr, W1, b1, W2, b2, W3, b3, Wp, bp, Wf1, bf1, Wf2, bf2):
    raise NotImplementedError("write your pallas kernel here")


# speedup vs baseline: 7.3404x; 7.3404x over previous
import functools

import jax
import jax.numpy as jnp
from jax import lax
from jax.experimental import pallas as pl
from jax.experimental.pallas import tpu as pltpu
from jax.experimental.pallas import tpu_sc as plsc

_N = 10000
_E = 320000
_NP = 10240
_RPT = 640
_EB = 80

_MESH = plsc.VectorSubcoreMesh(core_axis_name="c", subcore_axis_name="s")


def _fill(ref, value):
    if ref.ndim == 1:
        @pl.loop(0, ref.shape[0] // 16)
        def _(j):
            ref[pl.ds(j * 16, 16)] = jnp.full((16,), value, jnp.float32)
    else:
        rows, cols = ref.shape
        @pl.loop(0, rows)
        def _(i):
            @pl.loop(0, cols // 16)
            def _(j):
                ref[i, pl.ds(j * 16, 16)] = jnp.full((16,), value, jnp.float32)


def _make_deg_kernel():
    epw = _E // 32

    @functools.partial(
        pl.kernel,
        mesh=_MESH,
        out_type=jax.ShapeDtypeStruct((2, _NP), jnp.float32),
        scratch_types=[
            pltpu.VMEM((_EB,), jnp.int32),
            pltpu.VMEM((_EB,), jnp.float32),
            pltpu.VMEM((_RPT,), jnp.float32),
            pltpu.VMEM_SHARED((_NP,), jnp.float32),
        ],
    )
    def deg_kernel(dst_hbm, out_hbm, dst_v, ones_v, zbuf_v, acc_sh):
        cid = lax.axis_index("c")
        sid = lax.axis_index("s")
        _fill(ones_v, 1.0)
        _fill(zbuf_v, 0.0)
        pltpu.sync_copy(zbuf_v, acc_sh.at[pl.ds(sid * _RPT, _RPT)])
        plsc.subcore_barrier()
        base = (cid * 16 + sid) * epw

        @pl.loop(0, epw // _EB)
        def _(b):
            pltpu.sync_copy(dst_hbm.at[pl.ds(base + b * _EB, _EB)], dst_v)
            pltpu.sync_copy(ones_v, acc_sh.at[dst_v], add=True)

        plsc.subcore_barrier()
        pltpu.sync_copy(acc_sh.at[pl.ds(sid * _RPT, _RPT)],
                        out_hbm.at[cid, pl.ds(sid * _RPT, _RPT)])

    return deg_kernel


def _make_agg_kernel(num_chunks, edge_split):
    epw = _E // 32 if edge_split else _E // 16
    out_planes = 2 if edge_split else num_chunks
    per_core = 1 if edge_split else num_chunks // 2

    @functools.partial(
        pl.kernel,
        mesh=_MESH,
        out_type=jax.ShapeDtypeStruct((out_planes, _NP, 128), jnp.float32),
        scratch_types=[
            pltpu.VMEM((_EB,), jnp.int32),
            pltpu.VMEM((_EB,), jnp.int32),
            pltpu.VMEM((_EB, 128), jnp.float32),
            pltpu.VMEM((_EB, 128), jnp.float32),
            pltpu.VMEM_SHARED((_NP, 128), jnp.float32),
        ],
    )
    def agg_kernel(src_hbm, dst_hbm, table_hbm, out_hbm, src_v, dst_v,
                   rows_v, zero_v, acc_sh):
        cid = lax.axis_index("c")
        sid = lax.axis_index("s")
        _fill(zero_v, 0.0)

        for t in range(per_core):
            if edge_split:
                plane = cid
                table = table_hbm.at[0]
                base = (cid * 16 + sid) * epw
            else:
                plane = cid * per_core + t
                table = table_hbm.at[plane]
                base = sid * epw

            @pl.loop(0, _RPT // _EB)
            def _(k):
                pltpu.sync_copy(zero_v,
                                acc_sh.at[pl.ds(sid * _RPT + k * _EB, _EB)])

            plsc.subcore_barrier()

            @pl.loop(0, epw // _EB)
            def _(b):
                off = base + b * _EB
                pltpu.sync_copy(src_hbm.at[pl.ds(off, _EB)], src_v)
                pltpu.sync_copy(dst_hbm.at[pl.ds(off, _EB)], dst_v)
                pltpu.sync_copy(table.at[src_v], rows_v)
                pltpu.sync_copy(rows_v, acc_sh.at[dst_v], add=True)

            plsc.subcore_barrier()
            pltpu.sync_copy(acc_sh.at[pl.ds(sid * _RPT, _RPT)],
                            out_hbm.at[plane, pl.ds(sid * _RPT, _RPT)])

    return agg_kernel


_BLK = 1024
_GRID = _NP // _BLK


def _lrelu(v):
    return jnp.where(v >= 0, v, 0.15 * v)


def _dot(a, b):
    return jnp.dot(a, b, preferred_element_type=jnp.float32)


def _row_spec(shape):
    if len(shape) == 2:
        return pl.BlockSpec((_BLK, shape[1]), lambda i: (i, 0))
    return pl.BlockSpec((shape[0], _BLK, shape[2]), lambda i: (0, i, 0))


def _full_spec(shape):
    rank = len(shape)
    return pl.BlockSpec(shape, lambda i, _r=rank: (0,) * _r)


def _tc_call(body, row_ins, full_ins, out_shapes):
    in_specs = ([_row_spec(a.shape) for a in row_ins]
                + [_full_spec(a.shape) for a in full_ins])
    out_specs = [_row_spec(s) for s in out_shapes]
    return pl.pallas_call(
        body,
        grid=(_GRID,),
        in_specs=in_specs,
        out_specs=out_specs if len(out_shapes) > 1 else out_specs[0],
        out_shape=([jax.ShapeDtypeStruct(s, jnp.float32) for s in out_shapes]
                   if len(out_shapes) > 1
                   else jax.ShapeDtypeStruct(out_shapes[0], jnp.float32)),
    )(*row_ins, *full_ins)


def _prep_body(d0, d1, x_ref, w1t, q1_ref, dv_ref):
    deg = d0[...] + d1[...] + 1.0
    dis = lax.rsqrt(deg)
    dvb = jnp.broadcast_to(dis, (_BLK, 128))
    dv_ref[...] = dvb
    m1 = _dot(x_ref[...], w1t[...])
    for c in range(4):
        q1_ref[c] = dvb * m1[:, c * 128:(c + 1) * 128]


def _tc2_body(acc1, q1, dv, b1, w2ct, q2_ref):
    dvb = dv[...]
    h1 = jnp.concatenate([dvb * (acc1[c] + q1[c]) for c in range(4)], axis=1)
    h1 = _lrelu(h1 + b1[...])
    q2_ref[0] = dvb * _dot(h1, w2ct[0])
    q2_ref[1] = dvb * _dot(h1, w2ct[1])


def _tc3_body(acc2, q2, dv, b2, w3t, q3_ref):
    dvb = dv[...]
    a = jnp.concatenate([dvb * (acc2[0] + q2[0]), dvb * (acc2[1] + q2[1])],
                        axis=1)
    h2 = _lrelu(a + b2[...])
    q3_ref[...] = dvb * _dot(h2, w3t[...])


def _tc4_body(acc3, q3, dv, b3, wpt, bp, wf1t, bf1, wf2t, bf2, out_ref):
    h3 = _lrelu(dv[...] * (acc3[0] + acc3[1] + q3[...]) + b3[...])
    u1 = _dot(h3, wpt[...]) + bp[...]
    u2 = _lrelu(_dot(u1, wf1t[...]) + bf1[...])
    out_ref[...] = _dot(u2, wf2t[...]) + bf2[...]


def kernel(x, edge_index, edge_attr, W1, b1, W2, b2, W3, b3, Wp, bp,
           Wf1, bf1, Wf2, bf2):
    del edge_attr
    f32 = jnp.float32
    xp = jnp.pad(x, ((0, _NP - _N), (0, 0)))

    w1t = W1.T
    w2ct = W2.T.reshape(512, 2, 128).transpose(1, 0, 2)
    w3t = W3.T
    wf2p = jnp.zeros((128, 32), f32).at[:2].set(Wf2)
    bf2p = jnp.zeros((1, 128), f32).at[0, :2].set(bf2)

    deg_k = _make_deg_kernel()
    agg_split = _make_agg_kernel(1, True)
    agg_c2 = _make_agg_kernel(2, False)
    agg_c4 = _make_agg_kernel(4, False)

    e_src = edge_index[0]
    e_dst = edge_index[1]
    deg_parts = deg_k(e_dst)
    d0 = deg_parts[0].reshape(_NP, 1)
    d1 = deg_parts[1].reshape(_NP, 1)

    q1, dv = _tc_call(_prep_body, [d0, d1, xp], [w1t],
                      [(4, _NP, 128), (_NP, 128)])

    acc1 = agg_c4(e_src, e_dst, q1)
    q2 = _tc_call(_tc2_body, [acc1, q1, dv],
                  [b1.reshape(1, 512), w2ct], [(2, _NP, 128)])

    acc2 = agg_c2(e_src, e_dst, q2)
    q3 = _tc_call(_tc3_body, [acc2, q2, dv],
                  [b2.reshape(1, 256), w3t], [(_NP, 128)])

    acc3 = agg_split(e_src, e_dst, q3.reshape(1, _NP, 128))
    outp = _tc_call(_tc4_body, [acc3, q3, dv],
                    [b3.reshape(1, 128), Wp.T, bp.reshape(1, 16),
                     Wf1.T, bf1.reshape(1, 32), wf2p.T, bf2p],
                    [(_NP, 128)])
    return outp[:_N, :2]


# speedup vs baseline: 14.7925x; 2.0152x over previous
import functools

import jax
import jax.numpy as jnp
from jax import lax
from jax.experimental import pallas as pl
from jax.experimental.pallas import tpu as pltpu
from jax.experimental.pallas import tpu_sc as plsc

_N = 10000
_E = 320000
_NP = 10240
_RPT = 640
_EB = 80

_MESH = plsc.VectorSubcoreMesh(core_axis_name="c", subcore_axis_name="s")


def _fill(ref, value):
    if ref.ndim == 1:
        @pl.loop(0, ref.shape[0] // 16)
        def _(j):
            ref[pl.ds(j * 16, 16)] = jnp.full((16,), value, jnp.float32)
    else:
        rows, cols = ref.shape
        @pl.loop(0, rows)
        def _(i):
            @pl.loop(0, cols // 16)
            def _(j):
                ref[i, pl.ds(j * 16, 16)] = jnp.full((16,), value, jnp.float32)


def _make_deg_kernel():
    epw = _E // 32

    @functools.partial(
        pl.kernel,
        mesh=_MESH,
        out_type=jax.ShapeDtypeStruct((2, _NP), jnp.float32),
        scratch_types=[
            pltpu.VMEM((_EB,), jnp.int32),
            pltpu.VMEM((_EB,), jnp.float32),
            pltpu.VMEM((_RPT,), jnp.float32),
            pltpu.VMEM_SHARED((_NP,), jnp.float32),
        ],
    )
    def deg_kernel(dst_hbm, out_hbm, dst_v, ones_v, zbuf_v, acc_sh):
        cid = lax.axis_index("c")
        sid = lax.axis_index("s")
        _fill(ones_v, 1.0)
        _fill(zbuf_v, 0.0)
        pltpu.sync_copy(zbuf_v, acc_sh.at[pl.ds(sid * _RPT, _RPT)])
        plsc.subcore_barrier()
        base = (cid * 16 + sid) * epw

        @pl.loop(0, epw // _EB)
        def _(b):
            pltpu.sync_copy(dst_hbm.at[pl.ds(base + b * _EB, _EB)], dst_v)
            pltpu.sync_copy(ones_v, acc_sh.at[dst_v], add=True)

        plsc.subcore_barrier()
        pltpu.sync_copy(acc_sh.at[pl.ds(sid * _RPT, _RPT)],
                        out_hbm.at[cid, pl.ds(sid * _RPT, _RPT)])

    return deg_kernel


def _make_agg_kernel(num_chunks, edge_split):
    epw = _E // 32 if edge_split else _E // 16
    out_planes = 2 if edge_split else num_chunks
    per_core = 1 if edge_split else num_chunks // 2
    nb = epw // _EB

    @functools.partial(
        pl.kernel,
        mesh=_MESH,
        out_type=jax.ShapeDtypeStruct((out_planes, _NP, 128), jnp.float32),
        scratch_types=[
            pltpu.VMEM((_EB,), jnp.int32),
            pltpu.VMEM((_EB,), jnp.int32),
            pltpu.VMEM((_EB,), jnp.int32),
            pltpu.VMEM((_EB,), jnp.int32),
            pltpu.VMEM((_EB, 128), jnp.float32),
            pltpu.VMEM((_EB, 128), jnp.float32),
            pltpu.VMEM((_EB, 128), jnp.float32),
            pltpu.VMEM_SHARED((_NP, 128), jnp.float32),
            pltpu.SemaphoreType.DMA,
            pltpu.SemaphoreType.DMA,
            pltpu.SemaphoreType.DMA,
            pltpu.SemaphoreType.DMA,
        ],
    )
    def agg_kernel(src_hbm, dst_hbm, table_hbm, out_hbm, src0, src1,
                   dst0, dst1, rows0, rows1, zero_v, acc_sh,
                   si0, si1, sg0, sg1):
        cid = lax.axis_index("c")
        sid = lax.axis_index("s")
        srcs, dsts = (src0, src1), (dst0, dst1)
        rows, sis, sgs = (rows0, rows1), (si0, si1), (sg0, sg1)
        _fill(zero_v, 0.0)

        for t in range(per_core):
            if edge_split:
                plane = cid
                table = table_hbm.at[0]
                base = (cid * 16 + sid) * epw
            else:
                plane = cid * per_core + t
                table = table_hbm.at[plane]
                base = sid * epw

            def issue_idx(b, s):
                off = base + b * _EB
                pltpu.async_copy(src_hbm.at[pl.ds(off, _EB)], srcs[s],
                                 sis[s])
                pltpu.async_copy(dst_hbm.at[pl.ds(off, _EB)], dsts[s],
                                 sis[s])

            def wait_idx(s):
                pltpu.make_async_copy(src_hbm.at[pl.ds(base, _EB)],
                                      srcs[s], sis[s]).wait()
                pltpu.make_async_copy(dst_hbm.at[pl.ds(base, _EB)],
                                      dsts[s], sis[s]).wait()

            def issue_gather(s):
                pltpu.async_copy(table.at[srcs[s]], rows[s], sgs[s])

            def wait_gather(s):
                pltpu.make_async_copy(table.at[srcs[s]], rows[s],
                                      sgs[s]).wait()

            def scatter(s):
                pltpu.sync_copy(rows[s], acc_sh.at[dsts[s]], add=True)

            def full_iter(b, s):
                o = 1 - s
                wait_idx(s)
                issue_gather(s)
                wait_gather(o)
                scatter(o)
                issue_idx(b + 1, o)

            @pl.loop(0, _RPT // _EB)
            def _(k):
                pltpu.sync_copy(zero_v,
                                acc_sh.at[pl.ds(sid * _RPT + k * _EB, _EB)])

            plsc.subcore_barrier()

            issue_idx(0, 0)
            issue_idx(1, 1)
            wait_idx(0)
            issue_gather(0)

            pairs = (nb - 2) // 2

            @pl.loop(0, pairs)
            def _(g):
                b = 1 + 2 * g
                full_iter(b, 1)
                full_iter(b + 1, 0)

            if (nb - 2) % 2:
                full_iter(nb - 2, (nb - 2) % 2)

            s_last = (nb - 1) % 2
            o_last = 1 - s_last
            wait_idx(s_last)
            issue_gather(s_last)
            wait_gather(o_last)
            scatter(o_last)
            wait_gather(s_last)
            scatter(s_last)

            plsc.subcore_barrier()
            pltpu.sync_copy(acc_sh.at[pl.ds(sid * _RPT, _RPT)],
                            out_hbm.at[plane, pl.ds(sid * _RPT, _RPT)])

    return agg_kernel


_BLK = 1024
_GRID = _NP // _BLK


def _lrelu(v):
    return jnp.where(v >= 0, v, 0.15 * v)


def _dot(a, b):
    return jnp.dot(a, b, preferred_element_type=jnp.float32)


def _row_spec(shape):
    if len(shape) == 2:
        return pl.BlockSpec((_BLK, shape[1]), lambda i: (i, 0))
    return pl.BlockSpec((shape[0], _BLK, shape[2]), lambda i: (0, i, 0))


def _full_spec(shape):
    rank = len(shape)
    return pl.BlockSpec(shape, lambda i, _r=rank: (0,) * _r)


def _tc_call(body, row_ins, full_ins, out_shapes):
    in_specs = ([_row_spec(a.shape) for a in row_ins]
                + [_full_spec(a.shape) for a in full_ins])
    out_specs = [_row_spec(s) for s in out_shapes]
    return pl.pallas_call(
        body,
        grid=(_GRID,),
        in_specs=in_specs,
        out_specs=out_specs if len(out_shapes) > 1 else out_specs[0],
        out_shape=([jax.ShapeDtypeStruct(s, jnp.float32) for s in out_shapes]
                   if len(out_shapes) > 1
                   else jax.ShapeDtypeStruct(out_shapes[0], jnp.float32)),
    )(*row_ins, *full_ins)


def _prep_body(d0, d1, x_ref, w1t, q1_ref, dv_ref):
    deg = d0[...] + d1[...] + 1.0
    dis = lax.rsqrt(deg)
    dvb = jnp.broadcast_to(dis, (_BLK, 128))
    dv_ref[...] = dvb
    m1 = _dot(x_ref[...], w1t[...])
    for c in range(4):
        q1_ref[c] = dvb * m1[:, c * 128:(c + 1) * 128]


def _tc2_body(acc1, q1, dv, b1, w2ct, q2_ref):
    dvb = dv[...]
    h1 = jnp.concatenate([dvb * (acc1[c] + q1[c]) for c in range(4)], axis=1)
    h1 = _lrelu(h1 + b1[...])
    q2_ref[0] = dvb * _dot(h1, w2ct[0])
    q2_ref[1] = dvb * _dot(h1, w2ct[1])


def _tc3_body(acc2, q2, dv, b2, w3t, q3_ref):
    dvb = dv[...]
    a = jnp.concatenate([dvb * (acc2[0] + q2[0]), dvb * (acc2[1] + q2[1])],
                        axis=1)
    h2 = _lrelu(a + b2[...])
    q3_ref[...] = dvb * _dot(h2, w3t[...])


def _tc4_body(acc3, q3, dv, b3, wpt, bp, wf1t, bf1, wf2t, bf2, out_ref):
    h3 = _lrelu(dv[...] * (acc3[0] + acc3[1] + q3[...]) + b3[...])
    u1 = _dot(h3, wpt[...]) + bp[...]
    u2 = _lrelu(_dot(u1, wf1t[...]) + bf1[...])
    out_ref[...] = _dot(u2, wf2t[...]) + bf2[...]


def kernel(x, edge_index, edge_attr, W1, b1, W2, b2, W3, b3, Wp, bp,
           Wf1, bf1, Wf2, bf2):
    del edge_attr
    f32 = jnp.float32
    xp = jnp.pad(x, ((0, _NP - _N), (0, 0)))

    w1t = W1.T
    w2ct = W2.T.reshape(512, 2, 128).transpose(1, 0, 2)
    w3t = W3.T
    wf2p = jnp.zeros((128, 32), f32).at[:2].set(Wf2)
    bf2p = jnp.zeros((1, 128), f32).at[0, :2].set(bf2)

    deg_k = _make_deg_kernel()
    agg_split = _make_agg_kernel(1, True)
    agg_c2 = _make_agg_kernel(2, False)
    agg_c4 = _make_agg_kernel(4, False)

    e_src = edge_index[0]
    e_dst = edge_index[1]
    deg_parts = deg_k(e_dst)
    d0 = deg_parts[0].reshape(_NP, 1)
    d1 = deg_parts[1].reshape(_NP, 1)

    q1, dv = _tc_call(_prep_body, [d0, d1, xp], [w1t],
                      [(4, _NP, 128), (_NP, 128)])

    acc1 = agg_c4(e_src, e_dst, q1)
    q2 = _tc_call(_tc2_body, [acc1, q1, dv],
                  [b1.reshape(1, 512), w2ct], [(2, _NP, 128)])

    acc2 = agg_c2(e_src, e_dst, q2)
    q3 = _tc_call(_tc3_body, [acc2, q2, dv],
                  [b2.reshape(1, 256), w3t], [(_NP, 128)])

    acc3 = agg_split(e_src, e_dst, q3.reshape(1, _NP, 128))
    outp = _tc_call(_tc4_body, [acc3, q3, dv],
                    [b3.reshape(1, 128), Wp.T, bp.reshape(1, 16),
                     Wf1.T, bf1.reshape(1, 32), wf2p.T, bf2p],
                    [(_NP, 128)])
    return outp[:_N, :2]


# speedup vs baseline: 17.5542x; 1.1867x over previous
import functools

import jax
import jax.numpy as jnp
from jax import lax
from jax.experimental import pallas as pl
from jax.experimental.pallas import tpu as pltpu
from jax.experimental.pallas import tpu_sc as plsc

_N = 10000
_E = 320000
_NP = 10240
_RPT = 640
_EB = 80
_KB = 5

_MESH = plsc.VectorSubcoreMesh(core_axis_name="c", subcore_axis_name="s")


def _fill(ref, value):
    if ref.ndim == 1:
        @pl.loop(0, ref.shape[0] // 16)
        def _(j):
            ref[pl.ds(j * 16, 16)] = jnp.full((16,), value, jnp.float32)
    else:
        rows, cols = ref.shape
        @pl.loop(0, rows)
        def _(i):
            @pl.loop(0, cols // 16)
            def _(j):
                ref[i, pl.ds(j * 16, 16)] = jnp.full((16,), value, jnp.float32)


def _make_deg_kernel():
    epw = _E // 32

    @functools.partial(
        pl.kernel,
        mesh=_MESH,
        out_type=jax.ShapeDtypeStruct((2, _NP), jnp.float32),
        scratch_types=[
            pltpu.VMEM((_EB,), jnp.int32),
            pltpu.VMEM((_EB,), jnp.float32),
            pltpu.VMEM((_RPT,), jnp.float32),
            pltpu.VMEM_SHARED((_NP,), jnp.float32),
        ],
    )
    def deg_kernel(dst_hbm, out_hbm, dst_v, ones_v, zbuf_v, acc_sh):
        cid = lax.axis_index("c")
        sid = lax.axis_index("s")
        _fill(ones_v, 1.0)
        _fill(zbuf_v, 0.0)
        pltpu.sync_copy(zbuf_v, acc_sh.at[pl.ds(sid * _RPT, _RPT)])
        plsc.subcore_barrier()
        base = (cid * 16 + sid) * epw

        @pl.loop(0, epw // _EB)
        def _(b):
            pltpu.sync_copy(dst_hbm.at[pl.ds(base + b * _EB, _EB)], dst_v)
            pltpu.sync_copy(ones_v, acc_sh.at[dst_v], add=True)

        plsc.subcore_barrier()
        pltpu.sync_copy(acc_sh.at[pl.ds(sid * _RPT, _RPT)],
                        out_hbm.at[cid, pl.ds(sid * _RPT, _RPT)])

    return deg_kernel


def _make_agg_kernel(num_chunks, edge_split, kb):
    epw = _E // 32 if edge_split else _E // 16
    out_planes = 2 if edge_split else num_chunks
    per_core = 1 if edge_split else num_chunks // 2
    eb = kb * _EB
    nb = epw // eb
    zr = _RPT // 16 if kb > 1 else _EB

    @functools.partial(
        pl.kernel,
        mesh=_MESH,
        out_type=jax.ShapeDtypeStruct((out_planes, _NP, 128), jnp.float32),
        scratch_types=[
            pltpu.VMEM((eb,), jnp.int32),
            pltpu.VMEM((eb,), jnp.int32),
        ] + [pltpu.VMEM((_EB,), jnp.int32) for _ in range(2 * kb)] + [
            pltpu.VMEM((eb, 128), jnp.float32),
            pltpu.VMEM((eb, 128), jnp.float32),
            pltpu.VMEM((zr, 128), jnp.float32),
            pltpu.VMEM_SHARED((_NP, 128), jnp.float32),
            pltpu.SemaphoreType.DMA,
            pltpu.SemaphoreType.DMA,
            pltpu.SemaphoreType.DMA,
            pltpu.SemaphoreType.DMA,
        ],
    )
    def agg_kernel(src_hbm, dst_hbm, table_hbm, out_hbm, src0, src1,
                   *rest):
        dsts = (rest[:kb], rest[kb:2 * kb])
        (rows0, rows1, zero_v, acc_sh, si0, si1, sg0, sg1) = rest[2 * kb:]
        cid = lax.axis_index("c")
        sid = lax.axis_index("s")
        srcs = (src0, src1)
        rows, sis, sgs = (rows0, rows1), (si0, si1), (sg0, sg1)
        _fill(zero_v, 0.0)

        for t in range(per_core):
            if edge_split:
                plane = cid
                table = table_hbm.at[0]
                base = (cid * 16 + sid) * epw
            else:
                plane = cid * per_core + t
                table = table_hbm.at[plane]
                base = sid * epw

            def issue_idx(b, s):
                off = base + b * eb
                pltpu.async_copy(src_hbm.at[pl.ds(off, eb)], srcs[s],
                                 sis[s])
                for j in range(kb):
                    pltpu.async_copy(
                        dst_hbm.at[pl.ds(off + j * _EB, _EB)],
                        dsts[s][j], sis[s])

            def wait_idx(s):
                pltpu.make_async_copy(src_hbm.at[pl.ds(base, eb)],
                                      srcs[s], sis[s]).wait()
                for j in range(kb):
                    pltpu.make_async_copy(dst_hbm.at[pl.ds(base, _EB)],
                                          dsts[s][j], sis[s]).wait()

            def issue_gather(s):
                for j in range(kb):
                    pltpu.async_copy(
                        table.at[srcs[s].at[pl.ds(j * _EB, _EB)]],
                        rows[s].at[pl.ds(j * _EB, _EB)], sgs[s])

            def wait_gather(s):
                for j in range(kb):
                    pltpu.make_async_copy(
                        table.at[srcs[s].at[pl.ds(j * _EB, _EB)]],
                        rows[s].at[pl.ds(j * _EB, _EB)], sgs[s]).wait()

            def scatter(s):
                for j in range(kb):
                    pltpu.sync_copy(rows[s].at[pl.ds(j * _EB, _EB)],
                                    acc_sh.at[dsts[s][j]], add=True)

            def full_iter(b, s):
                o = 1 - s
                wait_idx(s)
                issue_gather(s)
                wait_gather(o)
                scatter(o)
                issue_idx(b + 1, o)

            @pl.loop(0, _RPT // zr)
            def _(k):
                pltpu.sync_copy(zero_v,
                                acc_sh.at[pl.ds(sid * _RPT + k * zr, zr)])

            plsc.subcore_barrier()

            issue_idx(0, 0)
            issue_idx(1, 1)
            wait_idx(0)
            issue_gather(0)

            pairs = (nb - 2) // 2

            @pl.loop(0, pairs)
            def _(g):
                b = 1 + 2 * g
                full_iter(b, 1)
                full_iter(b + 1, 0)

            if (nb - 2) % 2:
                full_iter(nb - 2, (nb - 2) % 2)

            s_last = (nb - 1) % 2
            o_last = 1 - s_last
            wait_idx(s_last)
            issue_gather(s_last)
            wait_gather(o_last)
            scatter(o_last)
            wait_gather(s_last)
            scatter(s_last)

            plsc.subcore_barrier()
            pltpu.sync_copy(acc_sh.at[pl.ds(sid * _RPT, _RPT)],
                            out_hbm.at[plane, pl.ds(sid * _RPT, _RPT)])

    return agg_kernel


_BLK = 1024
_GRID = _NP // _BLK


def _lrelu(v):
    return jnp.where(v >= 0, v, 0.15 * v)


def _dot(a, b):
    return jnp.dot(a, b, preferred_element_type=jnp.float32)


def _row_spec(shape):
    if len(shape) == 2:
        return pl.BlockSpec((_BLK, shape[1]), lambda i: (i, 0))
    return pl.BlockSpec((shape[0], _BLK, shape[2]), lambda i: (0, i, 0))


def _full_spec(shape):
    rank = len(shape)
    return pl.BlockSpec(shape, lambda i, _r=rank: (0,) * _r)


def _tc_call(body, row_ins, full_ins, out_shapes):
    in_specs = ([_row_spec(a.shape) for a in row_ins]
                + [_full_spec(a.shape) for a in full_ins])
    out_specs = [_row_spec(s) for s in out_shapes]
    return pl.pallas_call(
        body,
        grid=(_GRID,),
        in_specs=in_specs,
        out_specs=out_specs if len(out_shapes) > 1 else out_specs[0],
        out_shape=([jax.ShapeDtypeStruct(s, jnp.float32) for s in out_shapes]
                   if len(out_shapes) > 1
                   else jax.ShapeDtypeStruct(out_shapes[0], jnp.float32)),
    )(*row_ins, *full_ins)


def _prep_body(d0, d1, x_ref, w1t, q1_ref, dv_ref):
    deg = d0[...] + d1[...] + 1.0
    dis = lax.rsqrt(deg)
    dvb = jnp.broadcast_to(dis, (_BLK, 128))
    dv_ref[...] = dvb
    m1 = _dot(x_ref[...], w1t[...])
    for c in range(4):
        q1_ref[c] = dvb * m1[:, c * 128:(c + 1) * 128]


def _tc2_body(acc1, q1, dv, b1, w2ct, q2_ref):
    dvb = dv[...]
    h1 = jnp.concatenate([dvb * (acc1[c] + q1[c]) for c in range(4)], axis=1)
    h1 = _lrelu(h1 + b1[...])
    q2_ref[0] = dvb * _dot(h1, w2ct[0])
    q2_ref[1] = dvb * _dot(h1, w2ct[1])


def _tc3_body(acc2, q2, dv, b2, w3t, q3_ref):
    dvb = dv[...]
    a = jnp.concatenate([dvb * (acc2[0] + q2[0]), dvb * (acc2[1] + q2[1])],
                        axis=1)
    h2 = _lrelu(a + b2[...])
    q3_ref[...] = dvb * _dot(h2, w3t[...])


def _tc4_body(acc3, q3, dv, b3, wpt, bp, wf1t, bf1, wf2t, bf2, out_ref):
    h3 = _lrelu(dv[...] * (acc3[0] + acc3[1] + q3[...]) + b3[...])
    u1 = _dot(h3, wpt[...]) + bp[...]
    u2 = _lrelu(_dot(u1, wf1t[...]) + bf1[...])
    out_ref[...] = _dot(u2, wf2t[...]) + bf2[...]


def kernel(x, edge_index, edge_attr, W1, b1, W2, b2, W3, b3, Wp, bp,
           Wf1, bf1, Wf2, bf2):
    del edge_attr
    f32 = jnp.float32
    xp = jnp.pad(x, ((0, _NP - _N), (0, 0)))

    w1t = W1.T
    w2ct = W2.T.reshape(512, 2, 128).transpose(1, 0, 2)
    w3t = W3.T
    wf2p = jnp.zeros((128, 32), f32).at[:2].set(Wf2)
    bf2p = jnp.zeros((1, 128), f32).at[0, :2].set(bf2)

    deg_k = _make_deg_kernel()
    agg_split = _make_agg_kernel(1, True, 1)
    agg_c2 = _make_agg_kernel(2, False, 2)
    agg_c4 = _make_agg_kernel(4, False, 2)

    e_src = edge_index[0]
    e_dst = edge_index[1]
    deg_parts = deg_k(e_dst)
    d0 = deg_parts[0].reshape(_NP, 1)
    d1 = deg_parts[1].reshape(_NP, 1)

    q1, dv = _tc_call(_prep_body, [d0, d1, xp], [w1t],
                      [(4, _NP, 128), (_NP, 128)])

    acc1 = agg_c4(e_src, e_dst, q1)
    q2 = _tc_call(_tc2_body, [acc1, q1, dv],
                  [b1.reshape(1, 512), w2ct], [(2, _NP, 128)])

    acc2 = agg_c2(e_src, e_dst, q2)
    q3 = _tc_call(_tc3_body, [acc2, q2, dv],
                  [b2.reshape(1, 256), w3t], [(_NP, 128)])

    acc3 = agg_split(e_src, e_dst, q3.reshape(1, _NP, 128))
    outp = _tc_call(_tc4_body, [acc3, q3, dv],
                    [b3.reshape(1, 128), Wp.T, bp.reshape(1, 16),
                     Wf1.T, bf1.reshape(1, 32), wf2p.T, bf2p],
                    [(_NP, 128)])
    return outp[:_N, :2]


# speedup vs baseline: 18.7170x; 1.0662x over previous
import functools

import jax
import jax.numpy as jnp
from jax import lax
from jax.experimental import pallas as pl
from jax.experimental.pallas import tpu as pltpu
from jax.experimental.pallas import tpu_sc as plsc

_N = 10000
_E = 320000
_NP = 10240
_RPT = 640
_EB = 80
_KB = 5

_MESH = plsc.VectorSubcoreMesh(core_axis_name="c", subcore_axis_name="s")


def _fill(ref, value):
    if ref.ndim == 1:
        @pl.loop(0, ref.shape[0] // 16)
        def _(j):
            ref[pl.ds(j * 16, 16)] = jnp.full((16,), value, jnp.float32)
    else:
        rows, cols = ref.shape
        @pl.loop(0, rows)
        def _(i):
            @pl.loop(0, cols // 16)
            def _(j):
                ref[i, pl.ds(j * 16, 16)] = jnp.full((16,), value, jnp.float32)


def _make_deg_kernel():
    epw = _E // 32

    @functools.partial(
        pl.kernel,
        mesh=_MESH,
        out_type=jax.ShapeDtypeStruct((2, _NP), jnp.float32),
        scratch_types=[
            pltpu.VMEM((_EB,), jnp.int32),
            pltpu.VMEM((_EB,), jnp.int32),
            pltpu.VMEM((_EB,), jnp.float32),
            pltpu.VMEM((_RPT,), jnp.float32),
            pltpu.VMEM_SHARED((_NP,), jnp.float32),
            pltpu.SemaphoreType.DMA,
            pltpu.SemaphoreType.DMA,
        ],
    )
    def deg_kernel(dst_hbm, out_hbm, dst0, dst1, ones_v, zbuf_v, acc_sh,
                   si0, si1):
        cid = lax.axis_index("c")
        sid = lax.axis_index("s")
        dsts, sis = (dst0, dst1), (si0, si1)
        _fill(ones_v, 1.0)
        _fill(zbuf_v, 0.0)
        pltpu.sync_copy(zbuf_v, acc_sh.at[pl.ds(sid * _RPT, _RPT)])
        plsc.subcore_barrier()
        base = (cid * 16 + sid) * epw
        nb = epw // _EB

        def issue_idx(b, s):
            pltpu.async_copy(dst_hbm.at[pl.ds(base + b * _EB, _EB)],
                             dsts[s], sis[s])

        def step(b, s):
            issue_idx(b + 1, 1 - s)
            pltpu.make_async_copy(dst_hbm.at[pl.ds(base, _EB)],
                                  dsts[s], sis[s]).wait()
            pltpu.sync_copy(ones_v, acc_sh.at[dsts[s]], add=True)

        issue_idx(0, 0)

        @pl.loop(0, (nb - 1) // 2)
        def _(g):
            step(2 * g, 0)
            step(2 * g + 1, 1)

        if (nb - 1) % 2:
            step(nb - 2, (nb - 2) % 2)
        s_last = (nb - 1) % 2
        pltpu.make_async_copy(dst_hbm.at[pl.ds(base, _EB)],
                              dsts[s_last], sis[s_last]).wait()
        pltpu.sync_copy(ones_v, acc_sh.at[dsts[s_last]], add=True)

        plsc.subcore_barrier()
        pltpu.sync_copy(acc_sh.at[pl.ds(sid * _RPT, _RPT)],
                        out_hbm.at[cid, pl.ds(sid * _RPT, _RPT)])

    return deg_kernel


def _make_agg_kernel(num_chunks, edge_split, kb):
    epw = _E // 32 if edge_split else _E // 16
    out_planes = 2 if edge_split else num_chunks
    per_core = 1 if edge_split else num_chunks // 2
    eb = kb * _EB
    nb = epw // eb
    zr = _RPT // 16 if kb > 1 else _EB

    @functools.partial(
        pl.kernel,
        mesh=_MESH,
        out_type=jax.ShapeDtypeStruct((out_planes, _NP, 128), jnp.float32),
        scratch_types=[
            pltpu.VMEM((eb,), jnp.int32),
            pltpu.VMEM((eb,), jnp.int32),
        ] + [pltpu.VMEM((_EB,), jnp.int32) for _ in range(2 * kb)] + [
            pltpu.VMEM((eb, 128), jnp.float32),
            pltpu.VMEM((eb, 128), jnp.float32),
            pltpu.VMEM((zr, 128), jnp.float32),
            pltpu.VMEM_SHARED((_NP, 128), jnp.float32),
            pltpu.SemaphoreType.DMA,
            pltpu.SemaphoreType.DMA,
            pltpu.SemaphoreType.DMA,
            pltpu.SemaphoreType.DMA,
        ],
    )
    def agg_kernel(src_hbm, dst_hbm, table_hbm, out_hbm, src0, src1,
                   *rest):
        dsts = (rest[:kb], rest[kb:2 * kb])
        (rows0, rows1, zero_v, acc_sh, si0, si1, sg0, sg1) = rest[2 * kb:]
        cid = lax.axis_index("c")
        sid = lax.axis_index("s")
        srcs = (src0, src1)
        rows, sis, sgs = (rows0, rows1), (si0, si1), (sg0, sg1)
        _fill(zero_v, 0.0)

        for t in range(per_core):
            if edge_split:
                plane = cid
                table = table_hbm.at[0]
                base = (cid * 16 + sid) * epw
            else:
                plane = cid * per_core + t
                table = table_hbm.at[plane]
                base = sid * epw

            def issue_idx(b, s):
                off = base + b * eb
                pltpu.async_copy(src_hbm.at[pl.ds(off, eb)], srcs[s],
                                 sis[s])
                for j in range(kb):
                    pltpu.async_copy(
                        dst_hbm.at[pl.ds(off + j * _EB, _EB)],
                        dsts[s][j], sis[s])

            def wait_idx(s):
                pltpu.make_async_copy(src_hbm.at[pl.ds(base, eb)],
                                      srcs[s], sis[s]).wait()
                for j in range(kb):
                    pltpu.make_async_copy(dst_hbm.at[pl.ds(base, _EB)],
                                          dsts[s][j], sis[s]).wait()

            def issue_gather(s):
                for j in range(kb):
                    pltpu.async_copy(
                        table.at[srcs[s].at[pl.ds(j * _EB, _EB)]],
                        rows[s].at[pl.ds(j * _EB, _EB)], sgs[s])

            def wait_gather(s):
                for j in range(kb):
                    pltpu.make_async_copy(
                        table.at[srcs[s].at[pl.ds(j * _EB, _EB)]],
                        rows[s].at[pl.ds(j * _EB, _EB)], sgs[s]).wait()

            def scatter(s):
                for j in range(kb):
                    pltpu.sync_copy(rows[s].at[pl.ds(j * _EB, _EB)],
                                    acc_sh.at[dsts[s][j]], add=True)

            def full_iter(b, s):
                o = 1 - s
                wait_idx(s)
                issue_gather(s)
                wait_gather(o)
                scatter(o)
                issue_idx(b + 1, o)

            @pl.loop(0, _RPT // zr)
            def _(k):
                pltpu.sync_copy(zero_v,
                                acc_sh.at[pl.ds(sid * _RPT + k * zr, zr)])

            plsc.subcore_barrier()

            issue_idx(0, 0)
            issue_idx(1, 1)
            wait_idx(0)
            issue_gather(0)

            pairs = (nb - 2) // 2

            @pl.loop(0, pairs)
            def _(g):
                b = 1 + 2 * g
                full_iter(b, 1)
                full_iter(b + 1, 0)

            if (nb - 2) % 2:
                full_iter(nb - 2, (nb - 2) % 2)

            s_last = (nb - 1) % 2
            o_last = 1 - s_last
            wait_idx(s_last)
            issue_gather(s_last)
            wait_gather(o_last)
            scatter(o_last)
            wait_gather(s_last)
            scatter(s_last)

            if epw - nb * eb:
                off = base + nb * eb
                tsrc = srcs[0].at[pl.ds(0, _EB)]
                trows = rows[0].at[pl.ds(0, _EB)]
                pltpu.sync_copy(src_hbm.at[pl.ds(off, _EB)], tsrc)
                pltpu.sync_copy(dst_hbm.at[pl.ds(off, _EB)], dsts[0][0])
                pltpu.sync_copy(table.at[tsrc], trows)
                pltpu.sync_copy(trows, acc_sh.at[dsts[0][0]], add=True)

            plsc.subcore_barrier()
            pltpu.sync_copy(acc_sh.at[pl.ds(sid * _RPT, _RPT)],
                            out_hbm.at[plane, pl.ds(sid * _RPT, _RPT)])

    return agg_kernel


_BLK = 1024
_GRID = _NP // _BLK


def _lrelu(v):
    return jnp.where(v >= 0, v, 0.15 * v)


def _dot(a, b):
    return jnp.dot(a, b, preferred_element_type=jnp.float32)


def _row_spec(shape):
    if len(shape) == 2:
        return pl.BlockSpec((_BLK, shape[1]), lambda i: (i, 0))
    return pl.BlockSpec((shape[0], _BLK, shape[2]), lambda i: (0, i, 0))


def _full_spec(shape):
    rank = len(shape)
    return pl.BlockSpec(shape, lambda i, _r=rank: (0,) * _r)


def _tc_call(body, row_ins, full_ins, out_shapes):
    in_specs = ([_row_spec(a.shape) for a in row_ins]
                + [_full_spec(a.shape) for a in full_ins])
    out_specs = [_row_spec(s) for s in out_shapes]
    return pl.pallas_call(
        body,
        grid=(_GRID,),
        in_specs=in_specs,
        out_specs=out_specs if len(out_shapes) > 1 else out_specs[0],
        out_shape=([jax.ShapeDtypeStruct(s, jnp.float32) for s in out_shapes]
                   if len(out_shapes) > 1
                   else jax.ShapeDtypeStruct(out_shapes[0], jnp.float32)),
    )(*row_ins, *full_ins)


def _prep_body(d0, d1, x_ref, w1t, q1_ref, dv_ref):
    deg = d0[...] + d1[...] + 1.0
    dis = lax.rsqrt(deg)
    dvb = jnp.broadcast_to(dis, (_BLK, 128))
    dv_ref[...] = dvb
    m1 = _dot(x_ref[...], w1t[...])
    for c in range(4):
        q1_ref[c] = dvb * m1[:, c * 128:(c + 1) * 128]


def _tc2_body(acc1, q1, dv, b1, w2ct, q2_ref):
    dvb = dv[...]
    h1 = jnp.concatenate([dvb * (acc1[c] + q1[c]) for c in range(4)], axis=1)
    h1 = _lrelu(h1 + b1[...])
    q2_ref[0] = dvb * _dot(h1, w2ct[0])
    q2_ref[1] = dvb * _dot(h1, w2ct[1])


def _tc3_body(acc2, q2, dv, b2, w3t, q3_ref):
    dvb = dv[...]
    a = jnp.concatenate([dvb * (acc2[0] + q2[0]), dvb * (acc2[1] + q2[1])],
                        axis=1)
    h2 = _lrelu(a + b2[...])
    q3_ref[...] = dvb * _dot(h2, w3t[...])


def _tc4_body(acc3, q3, dv, b3, wpt, bp, wf1t, bf1, wf2t, bf2, out_ref):
    h3 = _lrelu(dv[...] * (acc3[0] + acc3[1] + q3[...]) + b3[...])
    u1 = _dot(h3, wpt[...]) + bp[...]
    u2 = _lrelu(_dot(u1, wf1t[...]) + bf1[...])
    out_ref[...] = _dot(u2, wf2t[...]) + bf2[...]


def kernel(x, edge_index, edge_attr, W1, b1, W2, b2, W3, b3, Wp, bp,
           Wf1, bf1, Wf2, bf2):
    del edge_attr
    f32 = jnp.float32
    xp = jnp.pad(x, ((0, _NP - _N), (0, 0)))

    w1t = W1.T
    w2ct = W2.T.reshape(512, 2, 128).transpose(1, 0, 2)
    w3t = W3.T
    wf2p = jnp.zeros((128, 32), f32).at[:2].set(Wf2)
    bf2p = jnp.zeros((1, 128), f32).at[0, :2].set(bf2)

    deg_k = _make_deg_kernel()
    agg_split = _make_agg_kernel(1, True, 2)
    agg_c2 = _make_agg_kernel(2, False, 2)
    agg_c4 = _make_agg_kernel(4, False, 2)

    e_src = edge_index[0]
    e_dst = edge_index[1]
    deg_parts = deg_k(e_dst)
    d0 = deg_parts[0].reshape(_NP, 1)
    d1 = deg_parts[1].reshape(_NP, 1)

    q1, dv = _tc_call(_prep_body, [d0, d1, xp], [w1t],
                      [(4, _NP, 128), (_NP, 128)])

    acc1 = agg_c4(e_src, e_dst, q1)
    q2 = _tc_call(_tc2_body, [acc1, q1, dv],
                  [b1.reshape(1, 512), w2ct], [(2, _NP, 128)])

    acc2 = agg_c2(e_src, e_dst, q2)
    q3 = _tc_call(_tc3_body, [acc2, q2, dv],
                  [b2.reshape(1, 256), w3t], [(_NP, 128)])

    acc3 = agg_split(e_src, e_dst, q3.reshape(1, _NP, 128))
    outp = _tc_call(_tc4_body, [acc3, q3, dv],
                    [b3.reshape(1, 128), Wp.T, bp.reshape(1, 16),
                     Wf1.T, bf1.reshape(1, 32), wf2p.T, bf2p],
                    [(_NP, 128)])
    return outp[:_N, :2]
